# Initial kernel scaffold; baseline (speedup 1.0000x reference)
import jax
import jax.numpy as jnp
from jax.experimental import pallas as pl


def kernel(positions, grid_main, grid_empty, occupancy, W0, b0, W1, b1, Wf, bf):
    raise NotImplementedError("write your pallas kernel here")


# speedup vs baseline: 5.3274x; 5.3274x over previous
import functools

import jax
import jax.numpy as jnp
from jax import lax
from jax.experimental import pallas as pl
from jax.experimental.pallas import tpu as pltpu
from jax.experimental.pallas import tpu_sc as plsc

N_PTS = 1048576
EMB = 16
RES = 128
G1 = RES + 1
G2 = G1 * G1
NW = 32
PTS_W = N_PTS // NW
BLK = 256
NBLK = PTS_W // BLK
NGRP = BLK // 16

ESC = 4.0
MAIN_SCALE = 128.0
MAIN_OFF = [a * G2 + b * G1 + z for a in (0, 1) for b in (0, 1) for z in (0, 1)]
EMPTY_OFF = [a * 400 + b * 80 + z * 16 for a in (0, 1) for b in (0, 1) for z in (0, 1)]


def _sc_body(pos_hbm, grid_hbm, occ_hbm, empty_hbm,
             emb_hbm, dx_hbm, dy_hbm, dz_hbm,
             posbuf, idxbuf, rowsbuf, featbuf, occbuf, embuf, sem):
    wid = lax.axis_index("s") * 2 + lax.axis_index("c")
    base_pt = wid * PTS_W

    pltpu.sync_copy(occ_hbm, occbuf)
    pltpu.sync_copy(empty_hbm, embuf)

    lanes = lax.iota(jnp.int32, 16)
    lanes3 = lanes * 3
    lanes8 = lanes * 8
    lanes16 = lanes * 16

    def main_cell(px, py, pz):
        ux = px * MAIN_SCALE
        uy = py * MAIN_SCALE
        uz = pz * MAIN_SCALE
        ix = jnp.clip(ux.astype(jnp.int32), 0, RES - 1)
        iy = jnp.clip(uy.astype(jnp.int32), 0, RES - 1)
        iz = jnp.clip(uz.astype(jnp.int32), 0, RES - 1)
        return ux, uy, uz, ix, iy, iz

    def load_pos(g):
        px = plsc.load_gather(posbuf, [lanes3 + (g * 48)])
        py = plsc.load_gather(posbuf, [lanes3 + (g * 48 + 1)])
        pz = plsc.load_gather(posbuf, [lanes3 + (g * 48 + 2)])
        return px, py, pz

    def blk_body(n, carry):
        p0 = base_pt + n * BLK
        pltpu.sync_copy(pos_hbm.at[pl.ds(p0 * 3, BLK * 3)], posbuf)

        def idx_body(g, c2):
            px, py, pz = load_pos(g)
            _, _, _, ix, iy, iz = main_cell(px, py, pz)
            base = ix * G2 + iy * G1 + iz
            gvec = jnp.full((16,), 0, jnp.int32) + g
            for c in range(8):
                plsc.store_scatter(idxbuf, [gvec, lanes8 + c], base + MAIN_OFF[c])
            return c2

        lax.fori_loop(0, NGRP, idx_body, 0, unroll=False)

        copies = []
        for g in range(NGRP):
            copies.append(pltpu.async_copy(
                grid_hbm.at[idxbuf.at[g]],
                rowsbuf.at[pl.ds(g * 128, 128)], sem))
        for cp in copies:
            cp.wait()

        def comb_body(g, c2):
            px, py, pz = load_pos(g)
            ux, uy, uz, ix, iy, iz = main_cell(px, py, pz)
            fx = ux - ix.astype(jnp.float32)
            fy = uy - iy.astype(jnp.float32)
            fz = uz - iz.astype(jnp.float32)

            ex = (px + 0.1) / 1.2 * ESC
            ey = (py + 0.1) / 1.2 * ESC
            ez = (pz + 0.1) / 1.2 * ESC
            jx = jnp.clip(ex.astype(jnp.int32), 0, 3)
            jy = jnp.clip(ey.astype(jnp.int32), 0, 3)
            jz = jnp.clip(ez.astype(jnp.int32), 0, 3)
            gx_f = ex - jx.astype(jnp.float32)
            gy_f = ey - jy.astype(jnp.float32)
            gz_f = ez - jz.astype(jnp.float32)
            ebase = jx * 400 + jy * 80 + jz * 16

            cx = jnp.clip((px * 32.0).astype(jnp.int32), 0, 31)
            cy = jnp.clip((py * 32.0).astype(jnp.int32), 0, 31)
            cz = jnp.clip((pz * 32.0).astype(jnp.int32), 0, 31)
            occv = plsc.load_gather(occbuf, [cx * 1024 + cy * 32 + cz])
            mask = occv > 0

            one = jnp.float32(1.0)
            wxm = (one - fx, fx)
            wym = (one - fy, fy)
            wzm = (one - fz, fz)
            wxe = (one - gx_f, gx_f)
            wye = (one - gy_f, gy_f)
            wze = (one - gz_f, gz_f)
            esc = jnp.float32(ESC / 1.2)
            msc = jnp.float32(MAIN_SCALE)

            wsel, xsel, ysel, zsel, rowv, eaddr = [], [], [], [], [], []
            for c in range(8):
                a, b, z = (c >> 2) & 1, (c >> 1) & 1, c & 1
                sa = 1.0 if a else -1.0
                sb = 1.0 if b else -1.0
                sz_ = 1.0 if z else -1.0
                wm = wxm[a] * wym[b] * wzm[z]
                we = wxe[a] * wye[b] * wze[z]
                dxm = (sa * msc) * (wym[b] * wzm[z])
                dxe = (sa * esc) * (wye[b] * wze[z])
                dym = (sb * msc) * (wxm[a] * wzm[z])
                dye = (sb * esc) * (wxe[a] * wze[z])
                dzm = (sz_ * msc) * (wxm[a] * wym[b])
                dze = (sz_ * esc) * (wxe[a] * wye[b])
                wsel.append(jnp.where(mask, wm, we))
                xsel.append(jnp.where(mask, dxm, dxe))
                ysel.append(jnp.where(mask, dym, dye))
                zsel.append(jnp.where(mask, dzm, dze))
                rowv.append(lanes8 + (g * 128 + c))
                eaddr.append(ebase + EMPTY_OFF[c])

            sbase = lanes16 + g * 256
            for f in range(EMB):
                colv = jnp.full((16,), 0, jnp.int32) + f
                acc_e = jnp.zeros((16,), jnp.float32)
                acc_x = jnp.zeros((16,), jnp.float32)
                acc_y = jnp.zeros((16,), jnp.float32)
                acc_z = jnp.zeros((16,), jnp.float32)
                for c in range(8):
                    vm = plsc.load_gather(rowsbuf, [rowv[c], colv])
                    ve = plsc.load_gather(embuf, [eaddr[c] + f])
                    v = jnp.where(mask, vm, ve)
                    acc_e = acc_e + wsel[c] * v
                    acc_x = acc_x + xsel[c] * v
                    acc_y = acc_y + ysel[c] * v
                    acc_z = acc_z + zsel[c] * v
                addr = sbase + f
                plsc.store_scatter(featbuf, [addr], acc_e)
                plsc.store_scatter(featbuf, [addr + 4096], acc_x)
                plsc.store_scatter(featbuf, [addr + 8192], acc_y)
                plsc.store_scatter(featbuf, [addr + 12288], acc_z)
            return c2

        lax.fori_loop(0, NGRP, comb_body, 0, unroll=False)

        nf = BLK * EMB
        pltpu.sync_copy(featbuf.at[pl.ds(0, nf)], emb_hbm.at[pl.ds(p0 * EMB, nf)])
        pltpu.sync_copy(featbuf.at[pl.ds(4096, nf)], dx_hbm.at[pl.ds(p0 * EMB, nf)])
        pltpu.sync_copy(featbuf.at[pl.ds(8192, nf)], dy_hbm.at[pl.ds(p0 * EMB, nf)])
        pltpu.sync_copy(featbuf.at[pl.ds(12288, nf)], dz_hbm.at[pl.ds(p0 * EMB, nf)])
        return carry

    lax.fori_loop(0, NBLK, blk_body, 0, unroll=False)


_plane = jax.ShapeDtypeStruct((N_PTS * EMB,), jnp.float32)

_sc_interp = functools.partial(
    pl.kernel,
    out_type=(_plane, _plane, _plane, _plane),
    mesh=plsc.VectorSubcoreMesh(core_axis_name="c", subcore_axis_name="s"),
    compiler_params=pltpu.CompilerParams(
        needs_layout_passes=False, use_tc_tiling_on_sc=False),
    scratch_types=[
        pltpu.VMEM((BLK * 3,), jnp.float32),
        pltpu.VMEM((NGRP, 128), jnp.int32),
        pltpu.VMEM((BLK * 8, EMB), jnp.float32),
        pltpu.VMEM((4 * BLK * EMB,), jnp.float32),
        pltpu.VMEM((32768,), jnp.int32),
        pltpu.VMEM((2000,), jnp.float32),
        pltpu.SemaphoreType.DMA,
    ],
)(_sc_body)


BT = 2048


def _tc_body(emb_ref, dx_ref, dy_ref, dz_ref, pos_ref,
             w0et_ref, w0pt_ref, w0e_ref, w0p_ref, b0_ref,
             w1t_ref, w1_ref, b1_ref, wft_ref, wf_ref, bf_ref,
             sdf_ref, grad_ref):
    emb = emb_ref[...]
    pos = pos_ref[...]
    a0 = (jnp.dot(emb, w0et_ref[...], preferred_element_type=jnp.float32)
          + b0_ref[...]
          + pos[:, 0:1] * w0pt_ref[0:1, :]
          + pos[:, 1:2] * w0pt_ref[1:2, :]
          + pos[:, 2:3] * w0pt_ref[2:3, :])
    h0 = jnp.sin(30.0 * a0)
    a1 = jnp.dot(h0, w1t_ref[...], preferred_element_type=jnp.float32) + b1_ref[...]
    h1 = jnp.sin(30.0 * a1)
    sdf_ref[...] = jnp.dot(h1, wft_ref[...], preferred_element_type=jnp.float32) + bf_ref[...]

    g_a1 = (30.0 * jnp.cos(30.0 * a1)) * wf_ref[...]
    g_h0 = jnp.dot(g_a1, w1_ref[...], preferred_element_type=jnp.float32)
    g_a0 = (30.0 * jnp.cos(30.0 * a0)) * g_h0
    g_emb = jnp.dot(g_a0, w0e_ref[...], preferred_element_type=jnp.float32)
    gd = jnp.dot(g_a0, w0p_ref[...], preferred_element_type=jnp.float32)
    gpx = jnp.sum(g_emb * dx_ref[...], axis=1, keepdims=True)
    gpy = jnp.sum(g_emb * dy_ref[...], axis=1, keepdims=True)
    gpz = jnp.sum(g_emb * dz_ref[...], axis=1, keepdims=True)
    grad_ref[...] = gd + jnp.concatenate([gpx, gpy, gpz], axis=1)


def _tc_call(emb_a, dx_a, dy_a, dz_a, positions,
             w0et, w0pt, w0e, w0p, b0r, w1t, w1, b1r, wft, wf, bfr):
    nblk = N_PTS // BT
    pt_spec = pl.BlockSpec((BT, EMB), lambda i: (i, 0))
    full = lambda s: pl.BlockSpec(s, lambda i: (0, 0))
    return pl.pallas_call(
        _tc_body,
        grid=(nblk,),
        in_specs=[
            pt_spec, pt_spec, pt_spec, pt_spec,
            pl.BlockSpec((BT, 3), lambda i: (i, 0)),
            full((EMB, 64)), full((3, 64)), full((64, EMB)), full((64, 3)),
            full((1, 64)), full((64, 64)), full((64, 64)), full((1, 64)),
            full((64, 1)), full((1, 64)), full((1, 1)),
        ],
        out_specs=[
            pl.BlockSpec((BT, 1), lambda i: (i, 0)),
            pl.BlockSpec((BT, 3), lambda i: (i, 0)),
        ],
        out_shape=[
            jax.ShapeDtypeStruct((N_PTS, 1), jnp.float32),
            jax.ShapeDtypeStruct((N_PTS, 3), jnp.float32),
        ],
        compiler_params=pltpu.CompilerParams(
            dimension_semantics=("arbitrary",)),
    )(emb_a, dx_a, dy_a, dz_a, positions,
      w0et, w0pt, w0e, w0p, b0r, w1t, w1, b1r, wft, wf, bfr)


def kernel(positions, grid_main, grid_empty, occupancy, W0, b0, W1, b1, Wf, bf):
    pos_flat = positions.reshape(-1)
    grid2d = grid_main.reshape(-1, EMB)
    occ_i32 = occupancy.reshape(-1).astype(jnp.int32)
    empty_flat = grid_empty.reshape(-1)

    emb_a, dx_a, dy_a, dz_a = _sc_interp(pos_flat, grid2d, occ_i32, empty_flat)
    emb_a = emb_a.reshape(N_PTS, EMB)
    dx_a = dx_a.reshape(N_PTS, EMB)
    dy_a = dy_a.reshape(N_PTS, EMB)
    dz_a = dz_a.reshape(N_PTS, EMB)

    w0e = W0[:, :EMB]
    w0p = W0[:, EMB:]
    sdf, grad = _tc_call(
        emb_a, dx_a, dy_a, dz_a, positions,
        w0e.T, w0p.T, w0e, w0p, b0.reshape(1, 64),
        W1.T, W1, b1.reshape(1, 64), Wf.T, Wf, bf.reshape(1, 1))
    return sdf, grad


# speedup vs baseline: 6.5877x; 1.2366x over previous
import functools

import jax
import jax.numpy as jnp
from jax import lax
from jax.experimental import pallas as pl
from jax.experimental.pallas import tpu as pltpu
from jax.experimental.pallas import tpu_sc as plsc

N_PTS = 1048576
EMB = 16
RES = 128
G1 = RES + 1
G2 = G1 * G1
NW = 32
PTS_W = N_PTS // NW
BLK = 256
NBLK = PTS_W // BLK
NGRP = BLK // 16
EROWS = 2048

ESC = 4.0
MAIN_SCALE = 128.0
MAIN_OFF = [a * G2 + b * G1 + z for a in (0, 1) for b in (0, 1) for z in (0, 1)]
EROW_OFF = [a * 25 + b * 5 + z for a in (0, 1) for b in (0, 1) for z in (0, 1)]


def _sc_body(pos_hbm, grid_hbm, occ_hbm, empty_hbm,
             emb_hbm, dx_hbm, dy_hbm, dz_hbm,
             posbuf, idxbuf, rowsbuf, featbuf, occbuf, sem):
    wid = lax.axis_index("s") * 2 + lax.axis_index("c")
    base_pt = wid * PTS_W

    pltpu.sync_copy(occ_hbm, occbuf)
    pltpu.sync_copy(empty_hbm, rowsbuf.at[pl.ds(EROWS, 125)])

    lanes = lax.iota(jnp.int32, 16)
    lanes3 = lanes * 3

    def load_pos(g):
        px = plsc.load_gather(posbuf, [lanes3 + (g * 48)])
        py = plsc.load_gather(posbuf, [lanes3 + (g * 48 + 1)])
        pz = plsc.load_gather(posbuf, [lanes3 + (g * 48 + 2)])
        return px, py, pz

    def main_cell(px, py, pz):
        ux = px * MAIN_SCALE
        uy = py * MAIN_SCALE
        uz = pz * MAIN_SCALE
        ix = jnp.clip(ux.astype(jnp.int32), 0, RES - 1)
        iy = jnp.clip(uy.astype(jnp.int32), 0, RES - 1)
        iz = jnp.clip(uz.astype(jnp.int32), 0, RES - 1)
        return ux, uy, uz, ix, iy, iz

    def blk_body(n, carry):
        p0 = base_pt + n * BLK
        pltpu.sync_copy(pos_hbm.at[pl.ds(p0 * 3, BLK * 3)], posbuf)

        def idx_body(g, c2):
            px, py, pz = load_pos(g)
            _, _, _, ix, iy, iz = main_cell(px, py, pz)
            base = ix * G2 + iy * G1 + iz
            gvec = jnp.zeros((16,), jnp.int32) + g
            for c in range(8):
                plsc.store_scatter(idxbuf, [gvec, lanes + (c * 16)],
                                   base + MAIN_OFF[c])
            return c2

        lax.fori_loop(0, NGRP, idx_body, 0, unroll=False)

        copies = []
        for g in range(NGRP):
            copies.append(pltpu.async_copy(
                grid_hbm.at[idxbuf.at[g]],
                rowsbuf.at[pl.ds(g * 128, 128)], sem))
        for cp in copies:
            cp.wait()

        def comb_body(g, c2):
            px, py, pz = load_pos(g)
            ux, uy, uz, ix, iy, iz = main_cell(px, py, pz)
            fx = ux - ix.astype(jnp.float32)
            fy = uy - iy.astype(jnp.float32)
            fz = uz - iz.astype(jnp.float32)

            ex = (px + 0.1) / 1.2 * ESC
            ey = (py + 0.1) / 1.2 * ESC
            ez = (pz + 0.1) / 1.2 * ESC
            jx = jnp.clip(ex.astype(jnp.int32), 0, 3)
            jy = jnp.clip(ey.astype(jnp.int32), 0, 3)
            jz = jnp.clip(ez.astype(jnp.int32), 0, 3)

            cx = jnp.clip((px * 32.0).astype(jnp.int32), 0, 31)
            cy = jnp.clip((py * 32.0).astype(jnp.int32), 0, 31)
            cz = jnp.clip((pz * 32.0).astype(jnp.int32), 0, 31)
            occv = plsc.load_gather(occbuf, [cx * 1024 + cy * 32 + cz])
            mask = occv > 0

            fsx = jnp.where(mask, fx, ex - jx.astype(jnp.float32))
            fsy = jnp.where(mask, fy, ey - jy.astype(jnp.float32))
            fsz = jnp.where(mask, fz, ez - jz.astype(jnp.float32))
            scl = jnp.where(mask, jnp.float32(MAIN_SCALE), jnp.float32(ESC / 1.2))
            re_base = (jx * 25 + jy * 5 + jz) + EROWS

            wx = (1.0 - fsx, fsx)
            wy = (1.0 - fsy, fsy)
            wz = (1.0 - fsz, fsz)

            wsel, xsel, ysel, zsel, rowv = [], [], [], [], []
            for c in range(8):
                a, b, z = (c >> 2) & 1, (c >> 1) & 1, c & 1
                sa = 1.0 if a else -1.0
                sb = 1.0 if b else -1.0
                sz_ = 1.0 if z else -1.0
                wyz = wy[b] * wz[z]
                wxz = wx[a] * wz[z]
                wxy = wx[a] * wy[b]
                wsel.append(wx[a] * wyz)
                xsel.append((sa * wyz) * scl)
                ysel.append((sb * wxz) * scl)
                zsel.append((sz_ * wxy) * scl)
                rm = lanes + (g * 128 + c * 16)
                rowv.append(jnp.where(mask, rm, re_base + EROW_OFF[c]))

            ptrow = lanes + (g * 16)
            rowe = ptrow + 256
            rowx = ptrow + 512
            rowz = ptrow + 768
            for f in range(EMB):
                colv = (lanes + f) & 15
                acc_e = jnp.zeros((16,), jnp.float32)
                acc_x = jnp.zeros((16,), jnp.float32)
                acc_y = jnp.zeros((16,), jnp.float32)
                acc_z = jnp.zeros((16,), jnp.float32)
                for c in range(8):
                    v = plsc.load_gather(rowsbuf, [rowv[c], colv])
                    acc_e = acc_e + wsel[c] * v
                    acc_x = acc_x + xsel[c] * v
                    acc_y = acc_y + ysel[c] * v
                    acc_z = acc_z + zsel[c] * v
                plsc.store_scatter(featbuf, [ptrow, colv], acc_e)
                plsc.store_scatter(featbuf, [rowe, colv], acc_x)
                plsc.store_scatter(featbuf, [rowx, colv], acc_y)
                plsc.store_scatter(featbuf, [rowz, colv], acc_z)
            return c2

        lax.fori_loop(0, NGRP, comb_body, 0, unroll=False)

        pltpu.sync_copy(featbuf.at[pl.ds(0, BLK)], emb_hbm.at[pl.ds(p0, BLK)])
        pltpu.sync_copy(featbuf.at[pl.ds(256, BLK)], dx_hbm.at[pl.ds(p0, BLK)])
        pltpu.sync_copy(featbuf.at[pl.ds(512, BLK)], dy_hbm.at[pl.ds(p0, BLK)])
        pltpu.sync_copy(featbuf.at[pl.ds(768, BLK)], dz_hbm.at[pl.ds(p0, BLK)])
        return carry

    lax.fori_loop(0, NBLK, blk_body, 0, unroll=False)


_plane = jax.ShapeDtypeStruct((N_PTS, EMB), jnp.float32)

_sc_interp = functools.partial(
    pl.kernel,
    out_type=(_plane, _plane, _plane, _plane),
    mesh=plsc.VectorSubcoreMesh(core_axis_name="c", subcore_axis_name="s"),
    compiler_params=pltpu.CompilerParams(
        needs_layout_passes=False, use_tc_tiling_on_sc=False),
    scratch_types=[
        pltpu.VMEM((BLK * 3,), jnp.float32),
        pltpu.VMEM((NGRP, 128), jnp.int32),
        pltpu.VMEM((EROWS + 125, EMB), jnp.float32),
        pltpu.VMEM((4 * BLK, EMB), jnp.float32),
        pltpu.VMEM((32768,), jnp.int32),
        pltpu.SemaphoreType.DMA,
    ],
)(_sc_body)


BT = 2048


def _tc_body(emb_ref, dx_ref, dy_ref, dz_ref, pos_ref,
             w0et_ref, w0pt_ref, w0e_ref, w0p_ref, b0_ref,
             w1t_ref, w1_ref, b1_ref, wft_ref, wf_ref, bf_ref,
             sdf_ref, grad_ref):
    emb = emb_ref[:, :EMB]
    pos = pos_ref[...]
    a0 = (jnp.dot(emb, w0et_ref[...], preferred_element_type=jnp.float32)
          + b0_ref[...]
          + pos[:, 0:1] * w0pt_ref[0:1, :]
          + pos[:, 1:2] * w0pt_ref[1:2, :]
          + pos[:, 2:3] * w0pt_ref[2:3, :])
    h0 = jnp.sin(30.0 * a0)
    a1 = jnp.dot(h0, w1t_ref[...], preferred_element_type=jnp.float32) + b1_ref[...]
    h1 = jnp.sin(30.0 * a1)
    sdf_ref[...] = jnp.dot(h1, wft_ref[...], preferred_element_type=jnp.float32) + bf_ref[...]

    g_a1 = (30.0 * jnp.cos(30.0 * a1)) * wf_ref[...]
    g_h0 = jnp.dot(g_a1, w1_ref[...], preferred_element_type=jnp.float32)
    g_a0 = (30.0 * jnp.cos(30.0 * a0)) * g_h0
    g_emb = jnp.dot(g_a0, w0e_ref[...], preferred_element_type=jnp.float32)
    gd = jnp.dot(g_a0, w0p_ref[...], preferred_element_type=jnp.float32)
    gpx = jnp.sum(g_emb * dx_ref[:, :EMB], axis=1, keepdims=True)
    gpy = jnp.sum(g_emb * dy_ref[:, :EMB], axis=1, keepdims=True)
    gpz = jnp.sum(g_emb * dz_ref[:, :EMB], axis=1, keepdims=True)
    grad_ref[...] = gd + jnp.concatenate([gpx, gpy, gpz], axis=1)


def _tc_call(emb_a, dx_a, dy_a, dz_a, positions,
             w0et, w0pt, w0e, w0p, b0r, w1t, w1, b1r, wft, wf, bfr):
    nblk = N_PTS // BT
    pt_spec = pl.BlockSpec((BT, EMB), lambda i: (i, 0))
    full = lambda s: pl.BlockSpec(s, lambda i: (0, 0))
    return pl.pallas_call(
        _tc_body,
        grid=(nblk,),
        in_specs=[
            pt_spec, pt_spec, pt_spec, pt_spec,
            pl.BlockSpec((BT, 3), lambda i: (i, 0)),
            full((EMB, 64)), full((3, 64)), full((64, EMB)), full((64, 3)),
            full((1, 64)), full((64, 64)), full((64, 64)), full((1, 64)),
            full((64, 1)), full((1, 64)), full((1, 1)),
        ],
        out_specs=[
            pl.BlockSpec((BT, 1), lambda i: (i, 0)),
            pl.BlockSpec((BT, 3), lambda i: (i, 0)),
        ],
        out_shape=[
            jax.ShapeDtypeStruct((N_PTS, 1), jnp.float32),
            jax.ShapeDtypeStruct((N_PTS, 3), jnp.float32),
        ],
        compiler_params=pltpu.CompilerParams(
            dimension_semantics=("arbitrary",)),
    )(emb_a, dx_a, dy_a, dz_a, positions,
      w0et, w0pt, w0e, w0p, b0r, w1t, w1, b1r, wft, wf, bfr)


def kernel(positions, grid_main, grid_empty, occupancy, W0, b0, W1, b1, Wf, bf):
    pos_flat = positions.reshape(-1)
    grid2d = grid_main.reshape(-1, EMB)
    occ_i32 = occupancy.reshape(-1).astype(jnp.int32)
    empty2d = grid_empty.reshape(-1, EMB)

    emb_a, dx_a, dy_a, dz_a = _sc_interp(pos_flat, grid2d, occ_i32, empty2d)

    w0e = W0[:, :EMB]
    w0p = W0[:, EMB:]
    sdf, grad = _tc_call(
        emb_a, dx_a, dy_a, dz_a, positions,
        w0e.T, w0p.T, w0e, w0p, b0.reshape(1, 64),
        W1.T, W1, b1.reshape(1, 64), Wf.T, Wf, bf.reshape(1, 1))
    return sdf, grad


# speedup vs baseline: 6.7796x; 1.0291x over previous
import functools

import jax
import jax.numpy as jnp
from jax import lax
from jax.experimental import pallas as pl
from jax.experimental.pallas import tpu as pltpu
from jax.experimental.pallas import tpu_sc as plsc

N_PTS = 1048576
EMB = 16
RES = 128
G1 = RES + 1
G2 = G1 * G1
NW = 32
PTS_W = N_PTS // NW
BLK = 256
NBLK = PTS_W // BLK
NGRP = BLK // 16
EROWS = 2048

ESC = 4.0
MAIN_SCALE = 128.0
MAIN_OFF = [a * G2 + b * G1 + z for a in (0, 1) for b in (0, 1) for z in (0, 1)]
EROW_OFF = [a * 25 + b * 5 + z for a in (0, 1) for b in (0, 1) for z in (0, 1)]


def _sc_body(pos_hbm, grid_hbm, occ_hbm, empty_hbm, feats_hbm,
             posbuf, idxbuf, rowsbuf, featbuf, occbuf, sem):
    wid = lax.axis_index("s") * 2 + lax.axis_index("c")
    base_pt = wid * PTS_W

    pltpu.sync_copy(occ_hbm, occbuf)
    pltpu.sync_copy(empty_hbm, rowsbuf.at[pl.ds(EROWS, 125)])

    lanes = lax.iota(jnp.int32, 16)
    lanes3 = lanes * 3

    def load_pos(g):
        px = plsc.load_gather(posbuf, [lanes3 + (g * 48)])
        py = plsc.load_gather(posbuf, [lanes3 + (g * 48 + 1)])
        pz = plsc.load_gather(posbuf, [lanes3 + (g * 48 + 2)])
        return px, py, pz

    def main_cell(px, py, pz):
        ux = px * MAIN_SCALE
        uy = py * MAIN_SCALE
        uz = pz * MAIN_SCALE
        ix = jnp.clip(ux.astype(jnp.int32), 0, RES - 1)
        iy = jnp.clip(uy.astype(jnp.int32), 0, RES - 1)
        iz = jnp.clip(uz.astype(jnp.int32), 0, RES - 1)
        return ux, uy, uz, ix, iy, iz

    def blk_body(n, carry):
        p0 = base_pt + n * BLK
        pltpu.sync_copy(pos_hbm.at[pl.ds(p0 * 3, BLK * 3)], posbuf)

        def idx_body(g, c2):
            px, py, pz = load_pos(g)
            _, _, _, ix, iy, iz = main_cell(px, py, pz)
            base = ix * G2 + iy * G1 + iz
            gvec = jnp.zeros((16,), jnp.int32) + g
            for c in range(8):
                plsc.store_scatter(idxbuf, [gvec, lanes + (c * 16)],
                                   base + MAIN_OFF[c])
            return c2

        lax.fori_loop(0, NGRP, idx_body, 0, unroll=False)

        copies = []
        for g in range(NGRP):
            copies.append(pltpu.async_copy(
                grid_hbm.at[idxbuf.at[g]],
                rowsbuf.at[pl.ds(g * 128, 128)], sem))
        for cp in copies:
            cp.wait()

        def comb_body(g, c2):
            px, py, pz = load_pos(g)
            ux, uy, uz, ix, iy, iz = main_cell(px, py, pz)
            fx = ux - ix.astype(jnp.float32)
            fy = uy - iy.astype(jnp.float32)
            fz = uz - iz.astype(jnp.float32)

            ex = (px + 0.1) / 1.2 * ESC
            ey = (py + 0.1) / 1.2 * ESC
            ez = (pz + 0.1) / 1.2 * ESC
            jx = jnp.clip(ex.astype(jnp.int32), 0, 3)
            jy = jnp.clip(ey.astype(jnp.int32), 0, 3)
            jz = jnp.clip(ez.astype(jnp.int32), 0, 3)

            cx = jnp.clip((px * 32.0).astype(jnp.int32), 0, 31)
            cy = jnp.clip((py * 32.0).astype(jnp.int32), 0, 31)
            cz = jnp.clip((pz * 32.0).astype(jnp.int32), 0, 31)
            occv = plsc.load_gather(occbuf, [cx * 1024 + cy * 32 + cz])
            mask = occv > 0

            fsx = jnp.where(mask, fx, ex - jx.astype(jnp.float32))
            fsy = jnp.where(mask, fy, ey - jy.astype(jnp.float32))
            fsz = jnp.where(mask, fz, ez - jz.astype(jnp.float32))
            scl = jnp.where(mask, jnp.float32(MAIN_SCALE), jnp.float32(ESC / 1.2))
            re_base = (jx * 25 + jy * 5 + jz) + EROWS

            wx = (1.0 - fsx, fsx)
            wy = (1.0 - fsy, fsy)
            wz = (1.0 - fsz, fsz)

            wsel, xsel, ysel, zsel, rowv = [], [], [], [], []
            for c in range(8):
                a, b, z = (c >> 2) & 1, (c >> 1) & 1, c & 1
                sa = 1.0 if a else -1.0
                sb = 1.0 if b else -1.0
                sz_ = 1.0 if z else -1.0
                wyz = wy[b] * wz[z]
                wxz = wx[a] * wz[z]
                wxy = wx[a] * wy[b]
                wsel.append(wx[a] * wyz)
                xsel.append((sa * wyz) * scl)
                ysel.append((sb * wxz) * scl)
                zsel.append((sz_ * wxy) * scl)
                rm = lanes + (g * 128 + c * 16)
                rowv.append(jnp.where(mask, rm, re_base + EROW_OFF[c]))

            ptrow = lanes + (g * 16)
            for f in range(EMB):
                colv = (lanes + f) & 15
                acc_e = jnp.zeros((16,), jnp.float32)
                acc_x = jnp.zeros((16,), jnp.float32)
                acc_y = jnp.zeros((16,), jnp.float32)
                acc_z = jnp.zeros((16,), jnp.float32)
                for c in range(8):
                    v = plsc.load_gather(rowsbuf, [rowv[c], colv])
                    acc_e = acc_e + wsel[c] * v
                    acc_x = acc_x + xsel[c] * v
                    acc_y = acc_y + ysel[c] * v
                    acc_z = acc_z + zsel[c] * v
                plsc.store_scatter(featbuf, [ptrow, colv], acc_e)
                plsc.store_scatter(featbuf, [ptrow, colv + 16], acc_x)
                plsc.store_scatter(featbuf, [ptrow, colv + 32], acc_y)
                plsc.store_scatter(featbuf, [ptrow, colv + 48], acc_z)
            return c2

        lax.fori_loop(0, NGRP, comb_body, 0, unroll=False)

        pltpu.sync_copy(featbuf, feats_hbm.at[pl.ds(p0, BLK)])
        return carry

    lax.fori_loop(0, NBLK, blk_body, 0, unroll=False)


_sc_interp = functools.partial(
    pl.kernel,
    out_type=jax.ShapeDtypeStruct((N_PTS, 64), jnp.float32),
    mesh=plsc.VectorSubcoreMesh(core_axis_name="c", subcore_axis_name="s"),
    compiler_params=pltpu.CompilerParams(
        needs_layout_passes=False, use_tc_tiling_on_sc=False),
    scratch_types=[
        pltpu.VMEM((BLK * 3,), jnp.float32),
        pltpu.VMEM((NGRP, 128), jnp.int32),
        pltpu.VMEM((EROWS + 125, EMB), jnp.float32),
        pltpu.VMEM((BLK, 64), jnp.float32),
        pltpu.VMEM((32768,), jnp.int32),
        pltpu.SemaphoreType.DMA,
    ],
)(_sc_body)


BT = 4096


def _tc_body(feats_ref, pos_ref,
             w0et_ref, w0pt_ref, w0e_ref, w0p_ref, b0_ref,
             w1t_ref, w1_ref, b1_ref, wft_ref, wf_ref, bf_ref,
             sdf_ref, grad_ref):
    feats = feats_ref[...]
    emb = feats[:, :EMB]
    pos = pos_ref[...]
    a0 = (jnp.dot(emb, w0et_ref[...], preferred_element_type=jnp.float32)
          + b0_ref[...]
          + pos[:, 0:1] * w0pt_ref[0:1, :]
          + pos[:, 1:2] * w0pt_ref[1:2, :]
          + pos[:, 2:3] * w0pt_ref[2:3, :])
    h0 = jnp.sin(30.0 * a0)
    a1 = jnp.dot(h0, w1t_ref[...], preferred_element_type=jnp.float32) + b1_ref[...]
    h1 = jnp.sin(30.0 * a1)
    sdf_ref[...] = jnp.dot(h1, wft_ref[...], preferred_element_type=jnp.float32) + bf_ref[...]

    g_a1 = (30.0 * jnp.cos(30.0 * a1)) * wf_ref[...]
    g_h0 = jnp.dot(g_a1, w1_ref[...], preferred_element_type=jnp.float32)
    g_a0 = (30.0 * jnp.cos(30.0 * a0)) * g_h0
    g_emb = jnp.dot(g_a0, w0e_ref[...], preferred_element_type=jnp.float32)
    gd = jnp.dot(g_a0, w0p_ref[...], preferred_element_type=jnp.float32)
    gpx = jnp.sum(g_emb * feats[:, 16:32], axis=1, keepdims=True)
    gpy = jnp.sum(g_emb * feats[:, 32:48], axis=1, keepdims=True)
    gpz = jnp.sum(g_emb * feats[:, 48:64], axis=1, keepdims=True)
    grad_ref[...] = gd + jnp.concatenate([gpx, gpy, gpz], axis=1)


def _tc_call(feats_a, positions,
             w0et, w0pt, w0e, w0p, b0r, w1t, w1, b1r, wft, wf, bfr):
    nblk = N_PTS // BT
    full = lambda s: pl.BlockSpec(s, lambda i: (0, 0))
    return pl.pallas_call(
        _tc_body,
        grid=(nblk,),
        in_specs=[
            pl.BlockSpec((BT, 64), lambda i: (i, 0)),
            pl.BlockSpec((BT, 3), lambda i: (i, 0)),
            full((EMB, 64)), full((3, 64)), full((64, EMB)), full((64, 3)),
            full((1, 64)), full((64, 64)), full((64, 64)), full((1, 64)),
            full((64, 1)), full((1, 64)), full((1, 1)),
        ],
        out_specs=[
            pl.BlockSpec((BT, 1), lambda i: (i, 0)),
            pl.BlockSpec((BT, 3), lambda i: (i, 0)),
        ],
        out_shape=[
            jax.ShapeDtypeStruct((N_PTS, 1), jnp.float32),
            jax.ShapeDtypeStruct((N_PTS, 3), jnp.float32),
        ],
        compiler_params=pltpu.CompilerParams(
            dimension_semantics=("arbitrary",)),
    )(feats_a, positions,
      w0et, w0pt, w0e, w0p, b0r, w1t, w1, b1r, wft, wf, bfr)


def kernel(positions, grid_main, grid_empty, occupancy, W0, b0, W1, b1, Wf, bf):
    pos_flat = positions.reshape(-1)
    grid2d = grid_main.reshape(-1, EMB)
    occ_i32 = occupancy.reshape(-1).astype(jnp.int32)
    empty2d = grid_empty.reshape(-1, EMB)

    feats_a = _sc_interp(pos_flat, grid2d, occ_i32, empty2d)

    w0e = W0[:, :EMB]
    w0p = W0[:, EMB:]
    sdf, grad = _tc_call(
        feats_a, positions,
        w0e.T, w0p.T, w0e, w0p, b0.reshape(1, 64),
        W1.T, W1, b1.reshape(1, 64), Wf.T, Wf, bf.reshape(1, 1))
    return sdf, grad


# speedup vs baseline: 10.1837x; 1.5021x over previous
import functools

import jax
import jax.numpy as jnp
from jax import lax
from jax.experimental import pallas as pl
from jax.experimental.pallas import tpu as pltpu
from jax.experimental.pallas import tpu_sc as plsc

N_PTS = 1048576
EMB = 16
RES = 128
G1 = RES + 1
G2 = G1 * G1
NW = 32
PTS_W = N_PTS // NW
BLK = 256
NBLK = PTS_W // BLK
NGRP = BLK // 16
EROWS = 2048

ESC = 4.0
MAIN_SCALE = 128.0


def _sc_body(pos_hbm, grid_hbm, occ_hbm, empty_hbm, feats_hbm,
             posbuf, idxbuf, rowsbuf, featbuf, occbuf, sem):
    wid = lax.axis_index("s") * 2 + lax.axis_index("c")
    base_pt = wid * PTS_W

    pltpu.sync_copy(occ_hbm, occbuf)
    pltpu.sync_copy(empty_hbm, rowsbuf.at[pl.ds(BLK, 64)])

    lanes = lax.iota(jnp.int32, 16)
    lanes3 = lanes * 3

    def load_pos(g):
        px = plsc.load_gather(posbuf, [lanes3 + (g * 48)])
        py = plsc.load_gather(posbuf, [lanes3 + (g * 48 + 1)])
        pz = plsc.load_gather(posbuf, [lanes3 + (g * 48 + 2)])
        return px, py, pz

    def main_cell(px, py, pz):
        ux = px * MAIN_SCALE
        uy = py * MAIN_SCALE
        uz = pz * MAIN_SCALE
        ix = jnp.clip(ux.astype(jnp.int32), 0, RES - 1)
        iy = jnp.clip(uy.astype(jnp.int32), 0, RES - 1)
        iz = jnp.clip(uz.astype(jnp.int32), 0, RES - 1)
        return ux, uy, uz, ix, iy, iz

    def blk_body(n, carry):
        p0 = base_pt + n * BLK
        pltpu.sync_copy(pos_hbm.at[pl.ds(p0 * 3, BLK * 3)], posbuf)

        def idx_body(g, c2):
            px, py, pz = load_pos(g)
            _, _, _, ix, iy, iz = main_cell(px, py, pz)
            cell = ix * 16384 + iy * 128 + iz
            gvec = jnp.zeros((16,), jnp.int32) + (g >> 3)
            plsc.store_scatter(idxbuf, [gvec, lanes + ((g & 7) * 16)], cell)
            return c2

        lax.fori_loop(0, NGRP, idx_body, 0, unroll=False)

        copies = []
        for g in range(2):
            copies.append(pltpu.async_copy(
                grid_hbm.at[idxbuf.at[g]],
                rowsbuf.at[pl.ds(g * 128, 128)], sem))
        for cp in copies:
            cp.wait()

        def comb_body(g, c2):
            px, py, pz = load_pos(g)
            ux, uy, uz, ix, iy, iz = main_cell(px, py, pz)
            fx = ux - ix.astype(jnp.float32)
            fy = uy - iy.astype(jnp.float32)
            fz = uz - iz.astype(jnp.float32)

            ex = (px + 0.1) / 1.2 * ESC
            ey = (py + 0.1) / 1.2 * ESC
            ez = (pz + 0.1) / 1.2 * ESC
            jx = jnp.clip(ex.astype(jnp.int32), 0, 3)
            jy = jnp.clip(ey.astype(jnp.int32), 0, 3)
            jz = jnp.clip(ez.astype(jnp.int32), 0, 3)

            cx = jnp.clip((px * 32.0).astype(jnp.int32), 0, 31)
            cy = jnp.clip((py * 32.0).astype(jnp.int32), 0, 31)
            cz = jnp.clip((pz * 32.0).astype(jnp.int32), 0, 31)
            occv = plsc.load_gather(occbuf, [cx * 1024 + cy * 32 + cz])
            mask = occv > 0

            fsx = jnp.where(mask, fx, ex - jx.astype(jnp.float32))
            fsy = jnp.where(mask, fy, ey - jy.astype(jnp.float32))
            fsz = jnp.where(mask, fz, ez - jz.astype(jnp.float32))
            scl = jnp.where(mask, jnp.float32(MAIN_SCALE), jnp.float32(ESC / 1.2))

            wx = (1.0 - fsx, fsx)
            wy = (1.0 - fsy, fsy)
            wz = (1.0 - fsz, fsz)

            ptrow = lanes + (g * 16)
            rowsel = jnp.where(mask, ptrow, (jx * 16 + jy * 4 + jz) + BLK)

            wsel, xsel, ysel, zsel = [], [], [], []
            for c in range(8):
                a, b, z = (c >> 2) & 1, (c >> 1) & 1, c & 1
                sa = 1.0 if a else -1.0
                sb = 1.0 if b else -1.0
                sz_ = 1.0 if z else -1.0
                wyz = wy[b] * wz[z]
                wxz = wx[a] * wz[z]
                wxy = wx[a] * wy[b]
                wsel.append(wx[a] * wyz)
                xsel.append((sa * wyz) * scl)
                ysel.append((sb * wxz) * scl)
                zsel.append((sz_ * wxy) * scl)

            for f in range(EMB):
                colv = (lanes + f) & 15
                acc_e = jnp.zeros((16,), jnp.float32)
                acc_x = jnp.zeros((16,), jnp.float32)
                acc_y = jnp.zeros((16,), jnp.float32)
                acc_z = jnp.zeros((16,), jnp.float32)
                for c in range(8):
                    v = plsc.load_gather(rowsbuf, [rowsel, colv + (c * 16)])
                    acc_e = acc_e + wsel[c] * v
                    acc_x = acc_x + xsel[c] * v
                    acc_y = acc_y + ysel[c] * v
                    acc_z = acc_z + zsel[c] * v
                plsc.store_scatter(featbuf, [ptrow, colv], acc_e)
                plsc.store_scatter(featbuf, [ptrow, colv + 16], acc_x)
                plsc.store_scatter(featbuf, [ptrow, colv + 32], acc_y)
                plsc.store_scatter(featbuf, [ptrow, colv + 48], acc_z)
            return c2

        lax.fori_loop(0, NGRP, comb_body, 0, unroll=False)

        pltpu.sync_copy(featbuf, feats_hbm.at[pl.ds(p0, BLK)])
        return carry

    lax.fori_loop(0, NBLK, blk_body, 0, unroll=False)


_sc_interp = functools.partial(
    pl.kernel,
    out_type=jax.ShapeDtypeStruct((N_PTS, 64), jnp.float32),
    mesh=plsc.VectorSubcoreMesh(core_axis_name="c", subcore_axis_name="s"),
    compiler_params=pltpu.CompilerParams(
        needs_layout_passes=False, use_tc_tiling_on_sc=False),
    scratch_types=[
        pltpu.VMEM((BLK * 3,), jnp.float32),
        pltpu.VMEM((2, 128), jnp.int32),
        pltpu.VMEM((BLK + 64, 128), jnp.float32),
        pltpu.VMEM((BLK, 64), jnp.float32),
        pltpu.VMEM((32768,), jnp.int32),
        pltpu.SemaphoreType.DMA,
    ],
)(_sc_body)


BT = 4096


def _tc_body(feats_ref, pos_ref,
             w0et_ref, w0pt_ref, w0e_ref, w0p_ref, b0_ref,
             w1t_ref, w1_ref, b1_ref, wft_ref, wf_ref, bf_ref,
             sdf_ref, grad_ref):
    feats = feats_ref[...]
    emb = feats[:, :EMB]
    pos = pos_ref[...]
    a0 = (jnp.dot(emb, w0et_ref[...], preferred_element_type=jnp.float32)
          + b0_ref[...]
          + pos[:, 0:1] * w0pt_ref[0:1, :]
          + pos[:, 1:2] * w0pt_ref[1:2, :]
          + pos[:, 2:3] * w0pt_ref[2:3, :])
    h0 = jnp.sin(30.0 * a0)
    a1 = jnp.dot(h0, w1t_ref[...], preferred_element_type=jnp.float32) + b1_ref[...]
    h1 = jnp.sin(30.0 * a1)
    sdf_ref[...] = jnp.dot(h1, wft_ref[...], preferred_element_type=jnp.float32) + bf_ref[...]

    g_a1 = (30.0 * jnp.cos(30.0 * a1)) * wf_ref[...]
    g_h0 = jnp.dot(g_a1, w1_ref[...], preferred_element_type=jnp.float32)
    g_a0 = (30.0 * jnp.cos(30.0 * a0)) * g_h0
    g_emb = jnp.dot(g_a0, w0e_ref[...], preferred_element_type=jnp.float32)
    gd = jnp.dot(g_a0, w0p_ref[...], preferred_element_type=jnp.float32)
    gpx = jnp.sum(g_emb * feats[:, 16:32], axis=1, keepdims=True)
    gpy = jnp.sum(g_emb * feats[:, 32:48], axis=1, keepdims=True)
    gpz = jnp.sum(g_emb * feats[:, 48:64], axis=1, keepdims=True)
    grad_ref[...] = gd + jnp.concatenate([gpx, gpy, gpz], axis=1)


def _tc_call(feats_a, positions,
             w0et, w0pt, w0e, w0p, b0r, w1t, w1, b1r, wft, wf, bfr):
    nblk = N_PTS // BT
    full = lambda s: pl.BlockSpec(s, lambda i: (0, 0))
    return pl.pallas_call(
        _tc_body,
        grid=(nblk,),
        in_specs=[
            pl.BlockSpec((BT, 64), lambda i: (i, 0)),
            pl.BlockSpec((BT, 3), lambda i: (i, 0)),
            full((EMB, 64)), full((3, 64)), full((64, EMB)), full((64, 3)),
            full((1, 64)), full((64, 64)), full((64, 64)), full((1, 64)),
            full((64, 1)), full((1, 64)), full((1, 1)),
        ],
        out_specs=[
            pl.BlockSpec((BT, 1), lambda i: (i, 0)),
            pl.BlockSpec((BT, 3), lambda i: (i, 0)),
        ],
        out_shape=[
            jax.ShapeDtypeStruct((N_PTS, 1), jnp.float32),
            jax.ShapeDtypeStruct((N_PTS, 3), jnp.float32),
        ],
        compiler_params=pltpu.CompilerParams(
            dimension_semantics=("arbitrary",)),
    )(feats_a, positions,
      w0et, w0pt, w0e, w0p, b0r, w1t, w1, b1r, wft, wf, bfr)


def _cell_table(grid, n):
    parts = [grid[a:a + n, b:b + n, z:z + n, :]
             for a in (0, 1) for b in (0, 1) for z in (0, 1)]
    return jnp.concatenate(parts, axis=-1).reshape(n * n * n, 128)


def kernel(positions, grid_main, grid_empty, occupancy, W0, b0, W1, b1, Wf, bf):
    pos_flat = positions.reshape(-1)
    grid_ct = _cell_table(grid_main, RES)
    occ_i32 = occupancy.reshape(-1).astype(jnp.int32)
    empty_ct = _cell_table(grid_empty, 4)

    feats_a = _sc_interp(pos_flat, grid_ct, occ_i32, empty_ct)

    w0e = W0[:, :EMB]
    w0p = W0[:, EMB:]
    sdf, grad = _tc_call(
        feats_a, positions,
        w0e.T, w0p.T, w0e, w0p, b0.reshape(1, 64),
        W1.T, W1, b1.reshape(1, 64), Wf.T, Wf, bf.reshape(1, 1))
    return sdf, grad


# speedup vs baseline: 13.5493x; 1.3305x over previous
import functools

import jax
import jax.numpy as jnp
from jax import lax
from jax.experimental import pallas as pl
from jax.experimental.pallas import tpu as pltpu
from jax.experimental.pallas import tpu_sc as plsc

N_PTS = 1048576
EMB = 16
RES = 128
G1 = RES + 1
G2 = G1 * G1
NW = 32
PTS_W = N_PTS // NW
BLK = 256
NBLK = PTS_W // BLK
NGRP = BLK // 16
EROWS = 2048

ESC = 4.0
MAIN_SCALE = 128.0


def _sc_body(pos_hbm, grid_hbm, occ_hbm, empty_hbm, feats_hbm,
             posbuf, idxbuf, rowsbuf, featbuf, occbuf, sem):
    wid = lax.axis_index("s") * 2 + lax.axis_index("c")
    base_pt = wid * PTS_W

    pltpu.sync_copy(occ_hbm, occbuf)
    pltpu.sync_copy(empty_hbm, rowsbuf.at[pl.ds(BLK, 64)])

    lanes = lax.iota(jnp.int32, 16)
    lanes3 = lanes * 3

    def load_pos(g):
        px = plsc.load_gather(posbuf, [lanes3 + (g * 48)])
        py = plsc.load_gather(posbuf, [lanes3 + (g * 48 + 1)])
        pz = plsc.load_gather(posbuf, [lanes3 + (g * 48 + 2)])
        return px, py, pz

    def main_cell(px, py, pz):
        ux = px * MAIN_SCALE
        uy = py * MAIN_SCALE
        uz = pz * MAIN_SCALE
        ix = jnp.clip(ux.astype(jnp.int32), 0, RES - 1)
        iy = jnp.clip(uy.astype(jnp.int32), 0, RES - 1)
        iz = jnp.clip(uz.astype(jnp.int32), 0, RES - 1)
        return ux, uy, uz, ix, iy, iz

    def blk_body(n, carry):
        p0 = base_pt + n * BLK
        pltpu.sync_copy(pos_hbm.at[pl.ds(p0 * 3, BLK * 3)], posbuf)

        def idx_body(g, c2):
            px, py, pz = load_pos(g)
            _, _, _, ix, iy, iz = main_cell(px, py, pz)
            cell = ix * 16384 + iy * 128 + iz
            gvec = jnp.zeros((16,), jnp.int32) + (g >> 3)
            plsc.store_scatter(idxbuf, [gvec, lanes + ((g & 7) * 16)], cell)
            return c2

        lax.fori_loop(0, NGRP, idx_body, 0, unroll=False)

        copies = []
        for g in range(2):
            copies.append(pltpu.async_copy(
                grid_hbm.at[idxbuf.at[g]],
                rowsbuf.at[pl.ds(g * 128, 128)], sem))
        for cp in copies:
            cp.wait()

        def comb_body(g, c2):
            px, py, pz = load_pos(g)
            ux, uy, uz, ix, iy, iz = main_cell(px, py, pz)
            fx = ux - ix.astype(jnp.float32)
            fy = uy - iy.astype(jnp.float32)
            fz = uz - iz.astype(jnp.float32)

            ex = (px + 0.1) / 1.2 * ESC
            ey = (py + 0.1) / 1.2 * ESC
            ez = (pz + 0.1) / 1.2 * ESC
            jx = jnp.clip(ex.astype(jnp.int32), 0, 3)
            jy = jnp.clip(ey.astype(jnp.int32), 0, 3)
            jz = jnp.clip(ez.astype(jnp.int32), 0, 3)

            cx = jnp.clip((px * 32.0).astype(jnp.int32), 0, 31)
            cy = jnp.clip((py * 32.0).astype(jnp.int32), 0, 31)
            cz = jnp.clip((pz * 32.0).astype(jnp.int32), 0, 31)
            occv = plsc.load_gather(occbuf, [cx * 1024 + cy * 32 + cz])
            mask = occv > 0

            fsx = jnp.where(mask, fx, ex - jx.astype(jnp.float32))
            fsy = jnp.where(mask, fy, ey - jy.astype(jnp.float32))
            fsz = jnp.where(mask, fz, ez - jz.astype(jnp.float32))
            scl = jnp.where(mask, jnp.float32(MAIN_SCALE), jnp.float32(ESC / 1.2))

            wx = (1.0 - fsx, fsx)
            wy = (1.0 - fsy, fsy)
            wz = (1.0 - fsz, fsz)

            ptrow = lanes + (g * 16)
            rowsel = jnp.where(mask, ptrow, (jx * 16 + jy * 4 + jz) + BLK)

            wsel, xsel, ysel, zsel = [], [], [], []
            for c in range(8):
                a, b, z = (c >> 2) & 1, (c >> 1) & 1, c & 1
                sa = 1.0 if a else -1.0
                sb = 1.0 if b else -1.0
                sz_ = 1.0 if z else -1.0
                wyz = wy[b] * wz[z]
                wxz = wx[a] * wz[z]
                wxy = wx[a] * wy[b]
                wsel.append(wx[a] * wyz)
                xsel.append((sa * wyz) * scl)
                ysel.append((sb * wxz) * scl)
                zsel.append((sz_ * wxy) * scl)

            for f in range(EMB):
                colv = (lanes + f) & 15
                acc_e = jnp.zeros((16,), jnp.float32)
                acc_x = jnp.zeros((16,), jnp.float32)
                acc_y = jnp.zeros((16,), jnp.float32)
                acc_z = jnp.zeros((16,), jnp.float32)
                for c in range(8):
                    v = plsc.load_gather(rowsbuf, [rowsel, colv + (c * 16)])
                    acc_e = acc_e + wsel[c] * v
                    acc_x = acc_x + xsel[c] * v
                    acc_y = acc_y + ysel[c] * v
                    acc_z = acc_z + zsel[c] * v
                plsc.store_scatter(featbuf, [ptrow, colv], acc_e)
                plsc.store_scatter(featbuf, [ptrow, colv + 16], acc_x)
                plsc.store_scatter(featbuf, [ptrow, colv + 32], acc_y)
                plsc.store_scatter(featbuf, [ptrow, colv + 48], acc_z)
            return c2

        lax.fori_loop(0, NGRP, comb_body, 0, unroll=False)

        pltpu.sync_copy(featbuf, feats_hbm.at[pl.ds(p0, BLK)])
        return carry

    lax.fori_loop(0, NBLK, blk_body, 0, unroll=False)


_sc_interp = functools.partial(
    pl.kernel,
    out_type=jax.ShapeDtypeStruct((N_PTS, 64), jnp.float32),
    mesh=plsc.VectorSubcoreMesh(core_axis_name="c", subcore_axis_name="s"),
    compiler_params=pltpu.CompilerParams(
        needs_layout_passes=False, use_tc_tiling_on_sc=False),
    scratch_types=[
        pltpu.VMEM((BLK * 3,), jnp.float32),
        pltpu.VMEM((2, 128), jnp.int32),
        pltpu.VMEM((BLK + 64, 128), jnp.float32),
        pltpu.VMEM((BLK, 64), jnp.float32),
        pltpu.VMEM((32768,), jnp.int32),
        pltpu.SemaphoreType.DMA,
    ],
)(_sc_body)


BT = 4096


def _fast_sin(x):
    k = jnp.floor(x * jnp.float32(1.0 / 3.14159265358979) + 0.5)
    y = (x - k * jnp.float32(3.14159274101)) + k * jnp.float32(8.742277657e-8)
    y2 = y * y
    p = jnp.float32(2.7525562e-6)
    p = p * y2 + jnp.float32(-1.9840874e-4)
    p = p * y2 + jnp.float32(8.3333293e-3)
    p = p * y2 + jnp.float32(-0.16666667)
    s = y + (y * y2) * p
    sbit = jnp.left_shift(jnp.bitwise_and(k.astype(jnp.int32), 1), 31)
    return jax.lax.bitcast_convert_type(
        jax.lax.bitcast_convert_type(s, jnp.int32) ^ sbit, jnp.float32)


def _fast_cos(x):
    return _fast_sin(x + jnp.float32(1.5707963267948966))


def _tc_body(feats_ref, pos_ref,
             w0et_ref, w0pt_ref, w0e_ref, w0p_ref, b0_ref,
             w1t_ref, w1_ref, b1_ref, wft_ref, wf_ref, bf_ref,
             sdf_ref, grad_ref):
    feats = feats_ref[...]
    emb = feats[:, :EMB]
    pos = pos_ref[...]
    a0 = (jnp.dot(emb, w0et_ref[...], preferred_element_type=jnp.float32)
          + b0_ref[...]
          + pos[:, 0:1] * w0pt_ref[0:1, :]
          + pos[:, 1:2] * w0pt_ref[1:2, :]
          + pos[:, 2:3] * w0pt_ref[2:3, :])
    arg0 = 30.0 * a0
    h0 = _fast_sin(arg0)
    a1 = jnp.dot(h0, w1t_ref[...], preferred_element_type=jnp.float32) + b1_ref[...]
    arg1 = 30.0 * a1
    h1 = _fast_sin(arg1)
    sdf_ref[...] = jnp.dot(h1, wft_ref[...], preferred_element_type=jnp.float32) + bf_ref[...]

    g_a1 = (30.0 * _fast_cos(arg1)) * wf_ref[...]
    g_h0 = jnp.dot(g_a1, w1_ref[...], preferred_element_type=jnp.float32)
    g_a0 = (30.0 * _fast_cos(arg0)) * g_h0
    g_emb = jnp.dot(g_a0, w0e_ref[...], preferred_element_type=jnp.float32)
    gd = jnp.dot(g_a0, w0p_ref[...], preferred_element_type=jnp.float32)
    gpx = jnp.sum(g_emb * feats[:, 16:32], axis=1, keepdims=True)
    gpy = jnp.sum(g_emb * feats[:, 32:48], axis=1, keepdims=True)
    gpz = jnp.sum(g_emb * feats[:, 48:64], axis=1, keepdims=True)
    grad_ref[...] = gd + jnp.concatenate([gpx, gpy, gpz], axis=1)


def _tc_call(feats_a, positions,
             w0et, w0pt, w0e, w0p, b0r, w1t, w1, b1r, wft, wf, bfr):
    nblk = N_PTS // BT
    full = lambda s: pl.BlockSpec(s, lambda i: (0, 0))
    return pl.pallas_call(
        _tc_body,
        grid=(nblk,),
        in_specs=[
            pl.BlockSpec((BT, 64), lambda i: (i, 0)),
            pl.BlockSpec((BT, 3), lambda i: (i, 0)),
            full((EMB, 64)), full((3, 64)), full((64, EMB)), full((64, 3)),
            full((1, 64)), full((64, 64)), full((64, 64)), full((1, 64)),
            full((64, 1)), full((1, 64)), full((1, 1)),
        ],
        out_specs=[
            pl.BlockSpec((BT, 1), lambda i: (i, 0)),
            pl.BlockSpec((BT, 3), lambda i: (i, 0)),
        ],
        out_shape=[
            jax.ShapeDtypeStruct((N_PTS, 1), jnp.float32),
            jax.ShapeDtypeStruct((N_PTS, 3), jnp.float32),
        ],
        compiler_params=pltpu.CompilerParams(
            dimension_semantics=("arbitrary",)),
    )(feats_a, positions,
      w0et, w0pt, w0e, w0p, b0r, w1t, w1, b1r, wft, wf, bfr)


def _cell_table(grid, n):
    parts = [grid[a:a + n, b:b + n, z:z + n, :]
             for a in (0, 1) for b in (0, 1) for z in (0, 1)]
    return jnp.concatenate(parts, axis=-1).reshape(n * n * n, 128)


def kernel(positions, grid_main, grid_empty, occupancy, W0, b0, W1, b1, Wf, bf):
    pos_flat = positions.reshape(-1)
    grid_ct = _cell_table(grid_main, RES)
    occ_i32 = occupancy.reshape(-1).astype(jnp.int32)
    empty_ct = _cell_table(grid_empty, 4)

    feats_a = _sc_interp(pos_flat, grid_ct, occ_i32, empty_ct)

    w0e = W0[:, :EMB]
    w0p = W0[:, EMB:]
    sdf, grad = _tc_call(
        feats_a, positions,
        w0e.T, w0p.T, w0e, w0p, b0.reshape(1, 64),
        W1.T, W1, b1.reshape(1, 64), Wf.T, Wf, bf.reshape(1, 1))
    return sdf, grad
